# baseline (speedup 1.0000x reference)
import jax
import jax.numpy as jnp
from jax import lax
from jax.experimental import pallas as pl
from jax.experimental.pallas import tpu as pltpu
from jax.experimental.pallas import tpu_sc as plsc

N_NODES = 10000
N_EDGES = 320000
FEAT = 128
HFEAT = FEAT // 2
BN_EPS_K = 1e-5

NC = 2
NS = 16
CHUNK_E = 128
CHUNKS = 160
CHUNKS_T = CHUNKS + 4
EDGES_PER_T = CHUNKS * CHUNK_E
E_PAD = NS * EDGES_PER_T
N_PAD = 10112
ROWS_PER_TILE = N_PAD // NS


def _sc_agg_body(h_hbm, src_hbm, dst_hbm, zero_hbm, out_hbm,
                 rows_a, rows_b, src_v, dst_v, sem_ga, sem_gb, acc_sh):
    cid = lax.axis_index("c")
    sid = lax.axis_index("s")

    pltpu.sync_copy(zero_hbm, acc_sh.at[pl.ds(sid * ROWS_PER_TILE, ROWS_PER_TILE)])
    pltpu.sync_copy(src_hbm.at[sid], src_v)
    pltpu.sync_copy(dst_hbm.at[sid], dst_v)
    plsc.subcore_barrier()

    h_half = h_hbm.at[cid]
    pltpu.async_copy(h_half.at[src_v.at[0]], rows_a, sem_ga)
    pltpu.async_copy(h_half.at[src_v.at[1]], rows_b, sem_gb)

    def loop_body(base, _):
        j = 2 * base
        pltpu.make_async_copy(h_half.at[src_v.at[j]], rows_a, sem_ga).wait()
        pltpu.sync_copy(rows_a, acc_sh.at[dst_v.at[j]], add=True)
        pltpu.async_copy(h_half.at[src_v.at[j + 2]], rows_a, sem_ga)
        pltpu.make_async_copy(h_half.at[src_v.at[j + 1]], rows_b, sem_gb).wait()
        pltpu.sync_copy(rows_b, acc_sh.at[dst_v.at[j + 1]], add=True)
        pltpu.async_copy(h_half.at[src_v.at[j + 3]], rows_b, sem_gb)
        return 0

    lax.fori_loop(0, (CHUNKS + 2) // 2, loop_body, 0)
    pltpu.make_async_copy(h_half.at[src_v.at[CHUNKS + 2]], rows_a, sem_ga).wait()
    pltpu.make_async_copy(h_half.at[src_v.at[CHUNKS + 3]], rows_b, sem_gb).wait()
    plsc.subcore_barrier()

    r0 = sid * ROWS_PER_TILE
    pltpu.sync_copy(acc_sh.at[pl.ds(r0, ROWS_PER_TILE)],
                    out_hbm.at[cid].at[pl.ds(r0, ROWS_PER_TILE)])


@jax.jit
def _sc_agg(hs, src_r, dst_r, zero_rows):
    mesh = plsc.VectorSubcoreMesh(core_axis_name="c", subcore_axis_name="s")
    fn = pl.kernel(
        _sc_agg_body,
        out_type=jax.ShapeDtypeStruct((NC, N_PAD, HFEAT), jnp.float32),
        mesh=mesh,
        compiler_params=pltpu.CompilerParams(use_tc_tiling_on_sc=False),
        scratch_types=[
            pltpu.VMEM((CHUNK_E, HFEAT), jnp.float32),
            pltpu.VMEM((CHUNK_E, HFEAT), jnp.float32),
            pltpu.VMEM((CHUNKS_T, CHUNK_E), jnp.int32),
            pltpu.VMEM((CHUNKS_T, CHUNK_E), jnp.int32),
            pltpu.SemaphoreType.DMA,
            pltpu.SemaphoreType.DMA,
            pltpu.VMEM_SHARED((N_PAD, HFEAT), jnp.float32),
        ],
    )
    return fn(hs, src_r, dst_r, zero_rows)


M_BLK = 1000
GRID_M = N_NODES // M_BLK


def _mlp2_body(xs_ref, p_ref, w1_ref, b1_ref, w2_ref, b2_ref, o_ref):
    h = (jnp.concatenate([xs_ref[0], xs_ref[1]], axis=1)
         + jnp.concatenate([p_ref[0], p_ref[1]], axis=1))
    h = jnp.maximum(jnp.dot(h, w1_ref[...], preferred_element_type=jnp.float32)
                    + b1_ref[...], 0.0)
    h = jnp.maximum(jnp.dot(h, w2_ref[...], preferred_element_type=jnp.float32)
                    + b2_ref[...], 0.0)
    o_ref[0] = h[:, :HFEAT]
    o_ref[1] = h[:, HFEAT:]


def _mlp3_body(hs_ref, q_ref, w1_ref, b1_ref, w2_ref, b2_ref,
               w3_ref, b3_ref, o_ref):
    h = (jnp.concatenate([hs_ref[0], hs_ref[1]], axis=1)
         + jnp.concatenate([q_ref[0], q_ref[1]], axis=1))
    h = jnp.maximum(jnp.dot(h, w1_ref[...], preferred_element_type=jnp.float32)
                    + b1_ref[...], 0.0)
    h = jnp.maximum(jnp.dot(h, w2_ref[...], preferred_element_type=jnp.float32)
                    + b2_ref[...], 0.0)
    o_ref[...] = jnp.dot(h, w3_ref[...], preferred_element_type=jnp.float32) + b3_ref[...]


def _split_spec():
    return pl.BlockSpec((NC, M_BLK, HFEAT), lambda i: (0, i, 0))


def _row_spec():
    return pl.BlockSpec((M_BLK, FEAT), lambda i: (i, 0))


def _full_spec(shape):
    return pl.BlockSpec(shape, lambda i: tuple(0 for _ in shape))


@jax.jit
def _tc_mlp2(xs, p, w1, b1, w2, b2):
    return pl.pallas_call(
        _mlp2_body,
        out_shape=jax.ShapeDtypeStruct((NC, N_NODES, HFEAT), jnp.float32),
        grid=(GRID_M,),
        in_specs=[_split_spec(), _split_spec(),
                  _full_spec((FEAT, FEAT)), _full_spec((1, FEAT)),
                  _full_spec((FEAT, FEAT)), _full_spec((1, FEAT))],
        out_specs=_split_spec(),
    )(xs, p, w1, b1, w2, b2)


@jax.jit
def _tc_mlp3(hs, q, w1, b1, w2, b2, w3, b3):
    return pl.pallas_call(
        _mlp3_body,
        out_shape=jax.ShapeDtypeStruct((N_NODES, FEAT), jnp.float32),
        grid=(GRID_M,),
        in_specs=[_split_spec(), _split_spec(),
                  _full_spec((FEAT, FEAT)), _full_spec((1, FEAT)),
                  _full_spec((FEAT, FEAT)), _full_spec((1, FEAT)),
                  _full_spec((FEAT, FEAT)), _full_spec((1, FEAT))],
        out_specs=_row_spec(),
    )(hs, q, w1, b1, w2, b2, w3, b3)


def _fold_bn(W, b, g, be, rm, rv):
    s = g / jnp.sqrt(rv + BN_EPS_K)
    wt = W.T * s[None, :]
    bf = ((b - rm) * s + be)[None, :]
    return wt, bf


def kernel(x, edge_index, W1, b1, W2, b2, W3, b3, W4, b4, W5, b5,
           g1, be1, rm1, rv1, g2, be2, rm2, rv2,
           g3, be3, rm3, rv3, g4, be4, rm4, rv4):
    n_fill = E_PAD - N_EDGES
    src_p = jnp.concatenate([edge_index[0], jnp.zeros((n_fill,), jnp.int32)])
    dst_p = jnp.concatenate(
        [edge_index[1],
         N_NODES + (jnp.arange(n_fill, dtype=jnp.int32) % (N_PAD - N_NODES))])
    src_r = jnp.concatenate(
        [src_p.reshape(NS, CHUNKS, CHUNK_E),
         jnp.zeros((NS, 4, CHUNK_E), jnp.int32)], axis=1)
    dst_r = jnp.concatenate(
        [dst_p.reshape(NS, CHUNKS, CHUNK_E),
         jnp.full((NS, 4, CHUNK_E), N_NODES, jnp.int32)], axis=1)
    zero_rows = jnp.zeros((ROWS_PER_TILE, HFEAT), jnp.float32)
    xs = jnp.stack([x[:, :HFEAT], x[:, HFEAT:]])

    w1t, b1f = _fold_bn(W1, b1, g1, be1, rm1, rv1)
    w2t, b2f = _fold_bn(W2, b2, g2, be2, rm2, rv2)
    w3t, b3f = _fold_bn(W3, b3, g3, be3, rm3, rv3)
    w4t, b4f = _fold_bn(W4, b4, g4, be4, rm4, rv4)
    w5t, b5f = W5.T, b5[None, :]

    p = _sc_agg(xs, src_r, dst_r, zero_rows)
    hs = _tc_mlp2(xs, p[:, :N_NODES], w1t, b1f, w2t, b2f)
    q = _sc_agg(hs, src_r, dst_r, zero_rows)
    out = _tc_mlp3(hs, q[:, :N_NODES], w3t, b3f, w4t, b4f, w5t, b5f)
    return out


# speedup vs baseline: 2.4804x; 2.4804x over previous
import jax
import jax.numpy as jnp
from jax import lax
from jax.experimental import pallas as pl
from jax.experimental.pallas import tpu as pltpu
from jax.experimental.pallas import tpu_sc as plsc

N_NODES = 10000
N_EDGES = 320000
FEAT = 128
BN_EPS_K = 1e-5

NC = 2
NS = 16
NW = NC * NS
EDGES_PER_W = N_EDGES // NW
CHUNK_E = 125
CHUNKS = EDGES_PER_W // CHUNK_E
N_PAD = 10112
ROWS_PER_TILE = N_PAD // NS


def _sc_agg_body(h_hbm, src_hbm, dst_hbm, zero_hbm, out_hbm,
                 rows_v, src_v, dst_v, sem_g, acc_sh):
    cid = lax.axis_index("c")
    sid = lax.axis_index("s")
    wid = sid * NC + cid

    pltpu.sync_copy(zero_hbm, acc_sh.at[pl.ds(sid * ROWS_PER_TILE, ROWS_PER_TILE)])
    pltpu.sync_copy(src_hbm.at[wid], src_v)
    pltpu.sync_copy(dst_hbm.at[wid], dst_v)
    plsc.subcore_barrier()

    def loop_body(j, _):
        pltpu.async_copy(h_hbm.at[src_v.at[j]], rows_v, sem_g).wait()
        pltpu.sync_copy(rows_v, acc_sh.at[dst_v.at[j]], add=True)
        return 0

    lax.fori_loop(0, CHUNKS, loop_body, 0)
    plsc.subcore_barrier()

    r0 = sid * ROWS_PER_TILE
    pltpu.sync_copy(acc_sh.at[pl.ds(r0, ROWS_PER_TILE)],
                    out_hbm.at[cid].at[pl.ds(r0, ROWS_PER_TILE)])


@jax.jit
def _sc_agg(h, src_r, dst_r, zero_rows):
    mesh = plsc.VectorSubcoreMesh(core_axis_name="c", subcore_axis_name="s")
    fn = pl.kernel(
        _sc_agg_body,
        out_type=jax.ShapeDtypeStruct((NC, N_PAD, FEAT), jnp.float32),
        mesh=mesh,
        scratch_types=[
            pltpu.VMEM((CHUNK_E, FEAT), jnp.float32),
            pltpu.VMEM((CHUNKS, CHUNK_E), jnp.int32),
            pltpu.VMEM((CHUNKS, CHUNK_E), jnp.int32),
            pltpu.SemaphoreType.DMA,
            pltpu.VMEM_SHARED((N_PAD, FEAT), jnp.float32),
        ],
    )
    return fn(h, src_r, dst_r, zero_rows)


M_BLK = 1000
GRID_M = N_NODES // M_BLK


def _mlp2_body(x_ref, p_ref, w1_ref, b1_ref, w2_ref, b2_ref, o_ref):
    h = x_ref[...] + p_ref[0] + p_ref[1]
    h = jnp.maximum(jnp.dot(h, w1_ref[...], preferred_element_type=jnp.float32)
                    + b1_ref[...], 0.0)
    h = jnp.maximum(jnp.dot(h, w2_ref[...], preferred_element_type=jnp.float32)
                    + b2_ref[...], 0.0)
    o_ref[...] = h


def _mlp3_body(x_ref, p_ref, w1_ref, b1_ref, w2_ref, b2_ref,
               w3_ref, b3_ref, o_ref):
    h = x_ref[...] + p_ref[0] + p_ref[1]
    h = jnp.maximum(jnp.dot(h, w1_ref[...], preferred_element_type=jnp.float32)
                    + b1_ref[...], 0.0)
    h = jnp.maximum(jnp.dot(h, w2_ref[...], preferred_element_type=jnp.float32)
                    + b2_ref[...], 0.0)
    o_ref[...] = jnp.dot(h, w3_ref[...], preferred_element_type=jnp.float32) + b3_ref[...]


def _row_spec():
    return pl.BlockSpec((M_BLK, FEAT), lambda i: (i, 0))


def _pad_spec():
    return pl.BlockSpec((NC, M_BLK, FEAT), lambda i: (0, i, 0))


def _full_spec(shape):
    return pl.BlockSpec(shape, lambda i: tuple(0 for _ in shape))


@jax.jit
def _tc_mlp2(x, p, w1, b1, w2, b2):
    return pl.pallas_call(
        _mlp2_body,
        out_shape=jax.ShapeDtypeStruct((N_NODES, FEAT), jnp.float32),
        grid=(GRID_M,),
        in_specs=[_row_spec(), _pad_spec(),
                  _full_spec((FEAT, FEAT)), _full_spec((1, FEAT)),
                  _full_spec((FEAT, FEAT)), _full_spec((1, FEAT))],
        out_specs=_row_spec(),
    )(x, p, w1, b1, w2, b2)


@jax.jit
def _tc_mlp3(x, p, w1, b1, w2, b2, w3, b3):
    return pl.pallas_call(
        _mlp3_body,
        out_shape=jax.ShapeDtypeStruct((N_NODES, FEAT), jnp.float32),
        grid=(GRID_M,),
        in_specs=[_row_spec(), _pad_spec(),
                  _full_spec((FEAT, FEAT)), _full_spec((1, FEAT)),
                  _full_spec((FEAT, FEAT)), _full_spec((1, FEAT)),
                  _full_spec((FEAT, FEAT)), _full_spec((1, FEAT))],
        out_specs=_row_spec(),
    )(x, p, w1, b1, w2, b2, w3, b3)


def _fold_bn(W, b, g, be, rm, rv):
    s = g / jnp.sqrt(rv + BN_EPS_K)
    wt = W.T * s[None, :]
    bf = ((b - rm) * s + be)[None, :]
    return wt, bf


def kernel(x, edge_index, W1, b1, W2, b2, W3, b3, W4, b4, W5, b5,
           g1, be1, rm1, rv1, g2, be2, rm2, rv2,
           g3, be3, rm3, rv3, g4, be4, rm4, rv4):
    src_r = edge_index[0].reshape(NW, CHUNKS, CHUNK_E)
    dst_r = edge_index[1].reshape(NW, CHUNKS, CHUNK_E)
    zero_rows = jnp.zeros((ROWS_PER_TILE, FEAT), jnp.float32)

    w1t, b1f = _fold_bn(W1, b1, g1, be1, rm1, rv1)
    w2t, b2f = _fold_bn(W2, b2, g2, be2, rm2, rv2)
    w3t, b3f = _fold_bn(W3, b3, g3, be3, rm3, rv3)
    w4t, b4f = _fold_bn(W4, b4, g4, be4, rm4, rv4)
    w5t, b5f = W5.T, b5[None, :]

    p = _sc_agg(x, src_r, dst_r, zero_rows)
    h = _tc_mlp2(x, p, w1t, b1f, w2t, b2f)
    q = _sc_agg(h, src_r, dst_r, zero_rows)
    out = _tc_mlp3(h, q, w3t, b3f, w4t, b4f, w5t, b5f)
    return out


# speedup vs baseline: 2.5066x; 1.0106x over previous
import jax
import jax.numpy as jnp
from jax import lax
from jax.experimental import pallas as pl
from jax.experimental.pallas import tpu as pltpu
from jax.experimental.pallas import tpu_sc as plsc

N_NODES = 10000
N_EDGES = 320000
FEAT = 128
BN_EPS_K = 1e-5

NC = 2
NS = 16
NW = NC * NS
EDGES_PER_W = N_EDGES // NW
CHUNK_E = 125
CHUNKS = EDGES_PER_W // CHUNK_E
N_PAD = 10112
ROWS_PER_TILE = N_PAD // NS


TAIL_REAL = N_NODES - (NS - 1) * ROWS_PER_TILE
TAIL_PAD = N_PAD - N_NODES


def _sc_agg_body(h_hbm, seed_hbm, zero_hbm, src_hbm, dst_hbm, out_hbm,
                 rows_v, src_v, dst_v, sem_g, acc_sh):
    cid = lax.axis_index("c")
    sid = lax.axis_index("s")
    wid = sid * NC + cid

    r0 = sid * ROWS_PER_TILE

    @pl.when(jnp.logical_and(cid == 0, sid < NS - 1))
    def _():
        pltpu.sync_copy(seed_hbm.at[pl.ds(r0, ROWS_PER_TILE)],
                        acc_sh.at[pl.ds(r0, ROWS_PER_TILE)])

    @pl.when(jnp.logical_and(cid == 0, sid == NS - 1))
    def _():
        t0 = (NS - 1) * ROWS_PER_TILE
        pltpu.sync_copy(seed_hbm.at[pl.ds(t0, TAIL_REAL)],
                        acc_sh.at[pl.ds(t0, TAIL_REAL)])
        pltpu.sync_copy(zero_hbm.at[pl.ds(0, TAIL_PAD)],
                        acc_sh.at[pl.ds(N_NODES, TAIL_PAD)])

    @pl.when(cid == 1)
    def _():
        pltpu.sync_copy(zero_hbm, acc_sh.at[pl.ds(r0, ROWS_PER_TILE)])
    pltpu.sync_copy(src_hbm.at[wid], src_v)
    pltpu.sync_copy(dst_hbm.at[wid], dst_v)
    plsc.subcore_barrier()

    def loop_body(j, _):
        pltpu.async_copy(h_hbm.at[src_v.at[j]], rows_v, sem_g).wait()
        pltpu.sync_copy(rows_v, acc_sh.at[dst_v.at[j]], add=True)
        return 0

    lax.fori_loop(0, CHUNKS, loop_body, 0)
    plsc.subcore_barrier()

    pltpu.sync_copy(acc_sh.at[pl.ds(r0, ROWS_PER_TILE)],
                    out_hbm.at[cid].at[pl.ds(r0, ROWS_PER_TILE)])


@jax.jit
def _sc_agg(h, seed, zero_rows, src_r, dst_r):
    mesh = plsc.VectorSubcoreMesh(core_axis_name="c", subcore_axis_name="s")
    fn = pl.kernel(
        _sc_agg_body,
        out_type=jax.ShapeDtypeStruct((NC, N_PAD, FEAT), jnp.float32),
        mesh=mesh,
        scratch_types=[
            pltpu.VMEM((CHUNK_E, FEAT), jnp.float32),
            pltpu.VMEM((CHUNKS, CHUNK_E), jnp.int32),
            pltpu.VMEM((CHUNKS, CHUNK_E), jnp.int32),
            pltpu.SemaphoreType.DMA,
            pltpu.VMEM_SHARED((N_PAD, FEAT), jnp.float32),
        ],
    )
    return fn(h, seed, zero_rows, src_r, dst_r)


M_BLK = 1000
GRID_M = N_NODES // M_BLK


def _mlp2_body(p_ref, w1_ref, b1_ref, w2_ref, b2_ref, o_ref):
    h = p_ref[0] + p_ref[1]
    h = jnp.maximum(jnp.dot(h, w1_ref[...], preferred_element_type=jnp.float32)
                    + b1_ref[...], 0.0)
    h = jnp.maximum(jnp.dot(h, w2_ref[...], preferred_element_type=jnp.float32)
                    + b2_ref[...], 0.0)
    o_ref[...] = h


def _mlp3_body(p_ref, w1_ref, b1_ref, w2_ref, b2_ref,
               w3_ref, b3_ref, o_ref):
    h = p_ref[0] + p_ref[1]
    h = jnp.maximum(jnp.dot(h, w1_ref[...], preferred_element_type=jnp.float32)
                    + b1_ref[...], 0.0)
    h = jnp.maximum(jnp.dot(h, w2_ref[...], preferred_element_type=jnp.float32)
                    + b2_ref[...], 0.0)
    o_ref[...] = jnp.dot(h, w3_ref[...], preferred_element_type=jnp.float32) + b3_ref[...]


def _row_spec():
    return pl.BlockSpec((M_BLK, FEAT), lambda i: (i, 0))


def _pad_spec():
    return pl.BlockSpec((NC, M_BLK, FEAT), lambda i: (0, i, 0))


def _full_spec(shape):
    return pl.BlockSpec(shape, lambda i: tuple(0 for _ in shape))


@jax.jit
def _tc_mlp2(p, w1, b1, w2, b2):
    return pl.pallas_call(
        _mlp2_body,
        out_shape=jax.ShapeDtypeStruct((N_NODES, FEAT), jnp.float32),
        grid=(GRID_M,),
        in_specs=[_pad_spec(),
                  _full_spec((FEAT, FEAT)), _full_spec((1, FEAT)),
                  _full_spec((FEAT, FEAT)), _full_spec((1, FEAT))],
        out_specs=_row_spec(),
    )(p, w1, b1, w2, b2)


@jax.jit
def _tc_mlp3(p, w1, b1, w2, b2, w3, b3):
    return pl.pallas_call(
        _mlp3_body,
        out_shape=jax.ShapeDtypeStruct((N_NODES, FEAT), jnp.float32),
        grid=(GRID_M,),
        in_specs=[_pad_spec(),
                  _full_spec((FEAT, FEAT)), _full_spec((1, FEAT)),
                  _full_spec((FEAT, FEAT)), _full_spec((1, FEAT)),
                  _full_spec((FEAT, FEAT)), _full_spec((1, FEAT))],
        out_specs=_row_spec(),
    )(p, w1, b1, w2, b2, w3, b3)


def _fold_bn(W, b, g, be, rm, rv):
    s = g / jnp.sqrt(rv + BN_EPS_K)
    wt = W.T * s[None, :]
    bf = ((b - rm) * s + be)[None, :]
    return wt, bf


def kernel(x, edge_index, W1, b1, W2, b2, W3, b3, W4, b4, W5, b5,
           g1, be1, rm1, rv1, g2, be2, rm2, rv2,
           g3, be3, rm3, rv3, g4, be4, rm4, rv4):
    src_r = edge_index[0].reshape(NW, CHUNKS, CHUNK_E)
    dst_r = edge_index[1].reshape(NW, CHUNKS, CHUNK_E)
    zero_rows = jnp.zeros((ROWS_PER_TILE, FEAT), jnp.float32)

    w1t, b1f = _fold_bn(W1, b1, g1, be1, rm1, rv1)
    w2t, b2f = _fold_bn(W2, b2, g2, be2, rm2, rv2)
    w3t, b3f = _fold_bn(W3, b3, g3, be3, rm3, rv3)
    w4t, b4f = _fold_bn(W4, b4, g4, be4, rm4, rv4)
    w5t, b5f = W5.T, b5[None, :]

    p = _sc_agg(x, x, zero_rows, src_r, dst_r)
    h = _tc_mlp2(p, w1t, b1f, w2t, b2f)
    q = _sc_agg(h, h, zero_rows, src_r, dst_r)
    out = _tc_mlp3(q, w3t, b3f, w4t, b4f, w5t, b5f)
    return out
